# baseline (speedup 1.0000x reference)
import functools

import jax
import jax.numpy as jnp
from jax.experimental import pallas as pl
from jax.experimental.pallas import tpu as pltpu

_IMG_W, _IMG_H = 1024, 800
_WIN_H = 8
_WIN_W = 16


def _roi_pool_kernel(coords_ref, f_ref, o_ref, *, n_rois, h, w):
    b = pl.program_id(0)
    neg = jnp.asarray(-jnp.inf, f_ref.dtype)
    zero = jnp.asarray(0.0, f_ref.dtype)
    rel_col = jax.lax.broadcasted_iota(jnp.int32, (_WIN_W, 1), 0)

    def body(n, carry):
        x1 = coords_ref[0, b, n]
        y1 = coords_ref[1, b, n]
        x2 = coords_ref[2, b, n]
        y2 = coords_ref[3, b, n]
        ys = jnp.minimum(y1, h - _WIN_H)
        xs = jnp.minimum((x1 // 8) * 8, w - _WIN_W)
        acc = None
        for r in range(_WIN_H):
            yr = ys + r
            bias = jnp.where((yr >= y1) & (yr < y2), zero, neg)
            row = f_ref[0, yr, pl.ds(xs, _WIN_W), :] + bias
            acc = row if acc is None else jnp.maximum(acc, row)
        cmask = (rel_col >= x2 - xs) | (rel_col < x1 - xs)
        acc = jnp.where(cmask, neg, acc)
        o_ref[0, pl.ds(n, 1), :] = jnp.max(acc, axis=0)[None, :]
        return carry

    jax.lax.fori_loop(0, 1, body, 0, unroll=4)


def kernel(features, roiss):
    B, C, H, W = features.shape
    N = roiss.shape[1]
    norm = roiss / jnp.array([_IMG_W, _IMG_H, _IMG_W, _IMG_H], dtype=roiss.dtype)
    x1 = jnp.clip(jnp.floor(norm[..., 0] * W).astype(jnp.int32), 0)
    y1 = jnp.clip(jnp.floor(norm[..., 1] * H).astype(jnp.int32), 0)
    x2 = jnp.clip(jnp.ceil(norm[..., 2] * W).astype(jnp.int32), 0)
    y2 = jnp.clip(jnp.ceil(norm[..., 3] * H).astype(jnp.int32), 0)
    x2 = jnp.where((x1 == 0) & (x2 == 0), x2 + 1, x2)
    y2 = jnp.where((y1 == 0) & (y2 == 0), y2 + 1, y2)
    x1 = jnp.where(x1 >= W, W - 1, x1)
    y1 = jnp.where(y1 >= H, H - 1, y1)
    coords = jnp.stack([x1, y1, x2, y2], axis=0)

    f = features.reshape(B, H, W, C)

    grid_spec = pltpu.PrefetchScalarGridSpec(
        num_scalar_prefetch=1,
        grid=(B,),
        in_specs=[pl.BlockSpec((1, H, W, C), lambda b, c: (b, 0, 0, 0))],
        out_specs=pl.BlockSpec((1, N, C), lambda b, c: (b, 0, 0)),
    )
    return pl.pallas_call(
        functools.partial(_roi_pool_kernel, n_rois=N, h=H, w=W),
        out_shape=jax.ShapeDtypeStruct((B, N, C), features.dtype),
        grid_spec=grid_spec,
        compiler_params=pltpu.CompilerParams(
            dimension_semantics=("arbitrary",),
        ),
        name="roi_max_pool",
    )(coords, f)


# speedup vs baseline: 2.3318x; 2.3318x over previous
import functools

import jax
import jax.numpy as jnp
from jax.experimental import pallas as pl
from jax.experimental.pallas import tpu as pltpu

_IMG_W, _IMG_H = 1024, 800
_WIN_H = 8
_WIN_W = 16


def _roi_pool_kernel(rtab_ref, ctab_ref, f_ref, o_ref, *, n_rois):
    b = pl.program_id(0)
    neg = jnp.asarray(-jnp.inf, f_ref.dtype)
    rel_col = jax.lax.broadcasted_iota(jnp.int32, (_WIN_W, 1), 0)

    def body(n, carry):
        acc = None
        for r in range(_WIN_H):
            roff = pl.multiple_of(rtab_ref[b, n, r], 8)
            row = f_ref[0, pl.ds(roff, _WIN_W), :]
            acc = row if acc is None else jnp.maximum(acc, row)
        cmask = (rel_col < ctab_ref[0, b, n]) | (rel_col >= ctab_ref[1, b, n])
        acc = jnp.where(cmask, neg, acc)
        o_ref[0, pl.ds(n, 1), :] = jnp.max(acc, axis=0)[None, :]
        return carry

    jax.lax.fori_loop(0, n_rois, body, 0, unroll=4)


def kernel(features, roiss):
    B, C, H, W = features.shape
    N = roiss.shape[1]
    norm = roiss / jnp.array([_IMG_W, _IMG_H, _IMG_W, _IMG_H], dtype=roiss.dtype)
    x1 = jnp.clip(jnp.floor(norm[..., 0] * W).astype(jnp.int32), 0)
    y1 = jnp.clip(jnp.floor(norm[..., 1] * H).astype(jnp.int32), 0)
    x2 = jnp.clip(jnp.ceil(norm[..., 2] * W).astype(jnp.int32), 0)
    y2 = jnp.clip(jnp.ceil(norm[..., 3] * H).astype(jnp.int32), 0)
    x2 = jnp.where((x1 == 0) & (x2 == 0), x2 + 1, x2)
    y2 = jnp.where((y1 == 0) & (y2 == 0), y2 + 1, y2)
    x1 = jnp.where(x1 >= W, W - 1, x1)
    y1 = jnp.where(y1 >= H, H - 1, y1)

    xs = jnp.minimum((x1 // 8) * 8, W - _WIN_W)
    hgt = jnp.minimum(y2, H) - y1
    r = jnp.arange(_WIN_H, dtype=jnp.int32)
    rtab = jnp.where(r[None, None, :] < hgt[..., None],
                     (y1[..., None] + r) * W + xs[..., None],
                     H * W)
    ctab = jnp.stack([x1 - xs, x2 - xs], axis=0)

    f = jnp.transpose(features, (0, 2, 3, 1)).reshape(B, H * W, C)
    f = jnp.pad(f, ((0, 0), (0, _WIN_W), (0, 0)), constant_values=-jnp.inf)

    grid_spec = pltpu.PrefetchScalarGridSpec(
        num_scalar_prefetch=2,
        grid=(B,),
        in_specs=[pl.BlockSpec((1, H * W + _WIN_W, C), lambda b, rt, ct: (b, 0, 0))],
        out_specs=pl.BlockSpec((1, N, C), lambda b, rt, ct: (b, 0, 0)),
    )
    return pl.pallas_call(
        functools.partial(_roi_pool_kernel, n_rois=N),
        out_shape=jax.ShapeDtypeStruct((B, N, C), features.dtype),
        grid_spec=grid_spec,
        compiler_params=pltpu.CompilerParams(
            dimension_semantics=("arbitrary",),
        ),
        name="roi_max_pool",
    )(rtab, ctab, f)


# speedup vs baseline: 4.1853x; 1.7949x over previous
import functools

import jax
import jax.numpy as jnp
from jax.experimental import pallas as pl
from jax.experimental.pallas import tpu as pltpu

_IMG_W, _IMG_H = 1024, 800
_WIN_H = 8
_WIN_W = 16


def _roi_pool_kernel(rtab_ref, ctab_ref, f_ref, o_ref, *, n_rois):
    b = pl.program_id(0)
    neg = jnp.asarray(-jnp.inf, f_ref.dtype)
    rel_col = jax.lax.broadcasted_iota(jnp.int32, (_WIN_W, 1), 0)

    def body(n, carry):
        acc = None
        for r in range(_WIN_H):
            roff = pl.multiple_of(rtab_ref[b, n, r], 8)
            row = f_ref[0, pl.ds(roff, _WIN_W), :]
            acc = row if acc is None else jnp.maximum(acc, row)
        cmask = (rel_col < ctab_ref[0, b, n]) | (rel_col >= ctab_ref[1, b, n])
        acc = jnp.where(cmask, neg, acc)
        o_ref[0, pl.ds(n, 1), :] = jnp.max(acc, axis=0)[None, :]
        return carry

    jax.lax.fori_loop(0, n_rois, body, 0, unroll=4)


def kernel(features, roiss):
    B, C, H, W = features.shape
    N = roiss.shape[1]
    norm = roiss / jnp.array([_IMG_W, _IMG_H, _IMG_W, _IMG_H], dtype=roiss.dtype)
    x1 = jnp.clip(jnp.floor(norm[..., 0] * W).astype(jnp.int32), 0)
    y1 = jnp.clip(jnp.floor(norm[..., 1] * H).astype(jnp.int32), 0)
    x2 = jnp.clip(jnp.ceil(norm[..., 2] * W).astype(jnp.int32), 0)
    y2 = jnp.clip(jnp.ceil(norm[..., 3] * H).astype(jnp.int32), 0)
    x2 = jnp.where((x1 == 0) & (x2 == 0), x2 + 1, x2)
    y2 = jnp.where((y1 == 0) & (y2 == 0), y2 + 1, y2)
    x1 = jnp.where(x1 >= W, W - 1, x1)
    y1 = jnp.where(y1 >= H, H - 1, y1)

    xs = jnp.minimum((x1 // 8) * 8, W - _WIN_W)
    hgt = jnp.minimum(y2, H) - y1
    r = jnp.arange(_WIN_H, dtype=jnp.int32)
    first = y1 * W + xs
    rtab = jnp.where(r[None, None, :] < hgt[..., None],
                     first[..., None] + r[None, None, :] * W,
                     first[..., None])
    ctab = jnp.stack([x1 - xs, x2 - xs], axis=0)

    f = jnp.transpose(features, (0, 2, 3, 1)).reshape(B, H * W, C)

    grid_spec = pltpu.PrefetchScalarGridSpec(
        num_scalar_prefetch=2,
        grid=(B,),
        in_specs=[pl.BlockSpec((1, H * W, C), lambda b, rt, ct: (b, 0, 0))],
        out_specs=pl.BlockSpec((1, N, C), lambda b, rt, ct: (b, 0, 0)),
    )
    return pl.pallas_call(
        functools.partial(_roi_pool_kernel, n_rois=N),
        out_shape=jax.ShapeDtypeStruct((B, N, C), features.dtype),
        grid_spec=grid_spec,
        compiler_params=pltpu.CompilerParams(
            dimension_semantics=("arbitrary",),
        ),
        name="roi_max_pool",
    )(rtab, ctab, f)


# speedup vs baseline: 4.4411x; 1.0611x over previous
import functools

import jax
import jax.numpy as jnp
from jax.experimental import pallas as pl
from jax.experimental.pallas import tpu as pltpu

_IMG_W, _IMG_H = 1024, 800
_WIN_H = 7
_WIN_W = 16


def _roi_pool_kernel(rtab_ref, ctab_ref, f_ref, o_ref, *, n_rois):
    b = pl.program_id(0)
    neg = jnp.asarray(-jnp.inf, f_ref.dtype)
    rel_col = jax.lax.broadcasted_iota(jnp.int32, (_WIN_W, 1), 0)

    def body(n, carry):
        acc = None
        for r in range(_WIN_H):
            roff = pl.multiple_of(rtab_ref[b, n, r], 8)
            row = f_ref[0, pl.ds(roff, _WIN_W), :]
            acc = row if acc is None else jnp.maximum(acc, row)
        cmask = (rel_col < ctab_ref[0, b, n]) | (rel_col >= ctab_ref[1, b, n])
        acc = jnp.where(cmask, neg, acc)
        o_ref[0, pl.ds(n, 1), :] = jnp.max(acc, axis=0)[None, :]
        return carry

    jax.lax.fori_loop(0, n_rois, body, 0, unroll=8)


def kernel(features, roiss):
    B, C, H, W = features.shape
    N = roiss.shape[1]
    norm = roiss / jnp.array([_IMG_W, _IMG_H, _IMG_W, _IMG_H], dtype=roiss.dtype)
    x1 = jnp.clip(jnp.floor(norm[..., 0] * W).astype(jnp.int32), 0)
    y1 = jnp.clip(jnp.floor(norm[..., 1] * H).astype(jnp.int32), 0)
    x2 = jnp.clip(jnp.ceil(norm[..., 2] * W).astype(jnp.int32), 0)
    y2 = jnp.clip(jnp.ceil(norm[..., 3] * H).astype(jnp.int32), 0)
    x2 = jnp.where((x1 == 0) & (x2 == 0), x2 + 1, x2)
    y2 = jnp.where((y1 == 0) & (y2 == 0), y2 + 1, y2)
    x1 = jnp.where(x1 >= W, W - 1, x1)
    y1 = jnp.where(y1 >= H, H - 1, y1)

    xs = jnp.minimum((x1 // 8) * 8, W - _WIN_W)
    hgt = jnp.minimum(y2, H) - y1
    r = jnp.arange(_WIN_H, dtype=jnp.int32)
    first = y1 * W + xs
    rtab = jnp.where(r[None, None, :] < hgt[..., None],
                     first[..., None] + r[None, None, :] * W,
                     first[..., None])
    ctab = jnp.stack([x1 - xs, x2 - xs], axis=0)

    f = jnp.transpose(features, (0, 2, 3, 1)).reshape(B, H * W, C)

    grid_spec = pltpu.PrefetchScalarGridSpec(
        num_scalar_prefetch=2,
        grid=(B,),
        in_specs=[pl.BlockSpec((1, H * W, C), lambda b, rt, ct: (b, 0, 0))],
        out_specs=pl.BlockSpec((1, N, C), lambda b, rt, ct: (b, 0, 0)),
    )
    return pl.pallas_call(
        functools.partial(_roi_pool_kernel, n_rois=N),
        out_shape=jax.ShapeDtypeStruct((B, N, C), features.dtype),
        grid_spec=grid_spec,
        compiler_params=pltpu.CompilerParams(
            dimension_semantics=("arbitrary",),
        ),
        name="roi_max_pool",
    )(rtab, ctab, f)


# speedup vs baseline: 4.5083x; 1.0151x over previous
import functools

import jax
import jax.numpy as jnp
from jax.experimental import pallas as pl
from jax.experimental.pallas import tpu as pltpu

_IMG_W, _IMG_H = 1024, 800
_WIN_H = 7
_WIN_W = 16


def _roi_pool_kernel(rtab_ref, ctab_ref, f_ref, o_ref, *, n_rois):
    b = pl.program_id(0)
    neg = jnp.asarray(-jnp.inf, f_ref.dtype)
    rel_col = jax.lax.broadcasted_iota(jnp.int32, (_WIN_W, 1), 0)

    def body(i, carry):
        nb = pl.multiple_of(i * 8, 8)
        for k in range(8):
            n = nb + k
            acc = None
            for r in range(_WIN_H):
                roff = pl.multiple_of(rtab_ref[b, n, r], 8)
                row = f_ref[0, pl.ds(roff, _WIN_W), :]
                acc = row if acc is None else jnp.maximum(acc, row)
            cmask = (rel_col < ctab_ref[0, b, n]) | (rel_col >= ctab_ref[1, b, n])
            acc = jnp.where(cmask, neg, acc)
            o_ref[0, pl.ds(n, 1), :] = jnp.max(acc, axis=0)[None, :]
        return carry

    jax.lax.fori_loop(0, n_rois // 8, body, 0, unroll=2)


def kernel(features, roiss):
    B, C, H, W = features.shape
    N = roiss.shape[1]
    norm = roiss / jnp.array([_IMG_W, _IMG_H, _IMG_W, _IMG_H], dtype=roiss.dtype)
    x1 = jnp.clip(jnp.floor(norm[..., 0] * W).astype(jnp.int32), 0)
    y1 = jnp.clip(jnp.floor(norm[..., 1] * H).astype(jnp.int32), 0)
    x2 = jnp.clip(jnp.ceil(norm[..., 2] * W).astype(jnp.int32), 0)
    y2 = jnp.clip(jnp.ceil(norm[..., 3] * H).astype(jnp.int32), 0)
    x2 = jnp.where((x1 == 0) & (x2 == 0), x2 + 1, x2)
    y2 = jnp.where((y1 == 0) & (y2 == 0), y2 + 1, y2)
    x1 = jnp.where(x1 >= W, W - 1, x1)
    y1 = jnp.where(y1 >= H, H - 1, y1)

    xs = jnp.minimum((x1 // 8) * 8, W - _WIN_W)
    hgt = jnp.minimum(y2, H) - y1
    r = jnp.arange(_WIN_H, dtype=jnp.int32)
    first = y1 * W + xs
    rtab = jnp.where(r[None, None, :] < hgt[..., None],
                     first[..., None] + r[None, None, :] * W,
                     first[..., None])
    rtab = jnp.pad(rtab, ((0, 0), (0, 0), (0, 8 - _WIN_H)))
    ctab = jnp.stack([x1 - xs, x2 - xs], axis=0)

    f = jnp.transpose(features, (0, 2, 3, 1)).reshape(B, H * W, C)

    grid_spec = pltpu.PrefetchScalarGridSpec(
        num_scalar_prefetch=2,
        grid=(B,),
        in_specs=[pl.BlockSpec((1, H * W, C), lambda b, rt, ct: (b, 0, 0))],
        out_specs=pl.BlockSpec((1, N, C), lambda b, rt, ct: (b, 0, 0)),
    )
    return pl.pallas_call(
        functools.partial(_roi_pool_kernel, n_rois=N),
        out_shape=jax.ShapeDtypeStruct((B, N, C), features.dtype),
        grid_spec=grid_spec,
        compiler_params=pltpu.CompilerParams(
            dimension_semantics=("arbitrary",),
        ),
        name="roi_max_pool",
    )(rtab, ctab, f)
